# baseline (speedup 1.0000x reference)
import functools

import jax
import jax.numpy as jnp
from jax import lax
from jax.experimental import pallas as pl
from jax.experimental.pallas import tpu as pltpu
from jax.experimental.pallas import tpu_sc as plsc

N_NODES = 100000
EMB_DIM = 128
PRIOR_DIM = 64
OUT_DIM = PRIOR_DIM + EMB_DIM
CHUNK = 256
HALF = 128
NW = 32

TC_BLOCK = 4096
SLAB = 12 * TC_BLOCK
SLAB2 = N_NODES - SLAB


def _build_gather(n_rows):
    num_full = n_rows // CHUNK
    rem = n_rows - num_full * CHUNK
    nmax = (num_full + NW - 1) // NW
    last_full_w = (num_full - 1) % NW
    mesh = plsc.VectorSubcoreMesh(core_axis_name="c", subcore_axis_name="s")

    @functools.partial(
        pl.kernel,
        mesh=mesh,
        out_type=jax.ShapeDtypeStruct((n_rows, EMB_DIM), jnp.float32),
        scratch_types=[
            pltpu.VMEM((CHUNK,), jnp.int32),
            pltpu.VMEM((CHUNK,), jnp.int32),
            pltpu.VMEM((CHUNK, EMB_DIM), jnp.float32),
            pltpu.VMEM((CHUNK, EMB_DIM), jnp.float32),
            pltpu.SemaphoreType.DMA,
            pltpu.SemaphoreType.DMA,
            pltpu.SemaphoreType.DMA,
            pltpu.SemaphoreType.DMA,
            pltpu.SemaphoreType.DMA,
            pltpu.SemaphoreType.DMA,
        ],
    )
    def k(feat_hbm, tab_hbm, emb_hbm,
          idx0, idx1, buf0, buf1, gs0, gs1, is0, is1, ws0, ws1):
        idx = (idx0, idx1)
        buf = (buf0, buf1)
        gsem = (gs0, gs1)
        isem = (is0, is1)
        wsem = (ws0, ws1)
        cid = lax.axis_index("c")
        sid = lax.axis_index("s")
        wid = sid * 2 + cid
        n = jnp.where(wid <= last_full_w, nmax, nmax - 1)

        def issue_gather(s):
            pltpu.async_copy(tab_hbm.at[idx[s].at[pl.ds(0, HALF)]],
                             buf[s].at[pl.ds(0, HALF), :], gsem[s])
            pltpu.async_copy(tab_hbm.at[idx[s].at[pl.ds(HALF, HALF)]],
                             buf[s].at[pl.ds(HALF, HALF), :], gsem[s])

        def wait_gather(s):
            pltpu.make_async_copy(tab_hbm.at[idx[s].at[pl.ds(0, HALF)]],
                                  buf[s].at[pl.ds(0, HALF), :],
                                  gsem[s]).wait()
            pltpu.make_async_copy(tab_hbm.at[idx[s].at[pl.ds(HALF, HALF)]],
                                  buf[s].at[pl.ds(HALF, HALF), :],
                                  gsem[s]).wait()

        def issue_idx(t, s):
            pltpu.async_copy(feat_hbm.at[pl.ds((wid + NW * t) * CHUNK, CHUNK)],
                             idx[s], isem[s])

        def wait_idx(s):
            pltpu.make_async_copy(feat_hbm.at[pl.ds(0, CHUNK)], idx[s],
                                  isem[s]).wait()

        def issue_write(t, s):
            c = wid + NW * t
            pltpu.async_copy(buf[s], emb_hbm.at[pl.ds(c * CHUNK, CHUNK), :],
                             wsem[s])

        def wait_write(s):
            pltpu.make_async_copy(buf[s], emb_hbm.at[pl.ds(0, CHUNK), :],
                                  wsem[s]).wait()

        pltpu.sync_copy(feat_hbm.at[pl.ds(wid * CHUNK, CHUNK)], idx[0])
        issue_gather(0)
        issue_idx(1, 1)

        def half(cur, t_cur):
            nxt = 1 - cur
            t_nxt = t_cur + 1

            @pl.when(t_nxt < n)
            def _():
                wait_idx(nxt)

                @pl.when(t_nxt >= 2)
                def _():
                    wait_write(nxt)

                issue_gather(nxt)

            @pl.when(t_cur < n)
            def _():
                wait_gather(cur)

                @pl.when(t_cur + 2 < n)
                def _():
                    issue_idx(t_cur + 2, cur)

                issue_write(t_cur, cur)

        def body(p, carry):
            half(0, 2 * p)
            half(1, 2 * p + 1)
            return carry

        lax.fori_loop(0, (nmax + 1) // 2, body, 0)

        wait_write(0)
        wait_write(1)

        if rem:
            r1 = min(rem, HALF)
            r2 = rem - r1

            @pl.when(wid == NW - 1)
            def _tail():
                base = num_full * CHUNK
                pltpu.sync_copy(feat_hbm.at[pl.ds(base, rem)],
                                idx[0].at[pl.ds(0, rem)])
                pltpu.async_copy(tab_hbm.at[idx[0].at[pl.ds(0, r1)]],
                                 buf[0].at[pl.ds(0, r1), :], gsem[0])
                pltpu.make_async_copy(tab_hbm.at[idx[0].at[pl.ds(0, r1)]],
                                      buf[0].at[pl.ds(0, r1), :],
                                      gsem[0]).wait()
                if r2:
                    pltpu.async_copy(tab_hbm.at[idx[0].at[pl.ds(r1, r2)]],
                                     buf[0].at[pl.ds(r1, r2), :], gsem[0])
                    pltpu.make_async_copy(
                        tab_hbm.at[idx[0].at[pl.ds(r1, r2)]],
                        buf[0].at[pl.ds(r1, r2), :], gsem[0]).wait()
                pltpu.sync_copy(buf[0].at[pl.ds(0, rem), :],
                                emb_hbm.at[pl.ds(base, rem), :])

    return k


_GATHER1 = _build_gather(SLAB)
_GATHER2 = _build_gather(SLAB2)


def _concat_body1(inv_ref, emb_ref, out_ref):
    out_ref[:PRIOR_DIM, :] = inv_ref[...]
    out_ref[PRIOR_DIM:, :] = emb_ref[...].T


def _concat_body2(init_ref, inv_ref, emb_ref, out_ref):
    del init_ref
    out_ref[:PRIOR_DIM, :] = inv_ref[...]
    out_ref[PRIOR_DIM:, :] = emb_ref[...].T


_CONCAT1 = pl.pallas_call(
    _concat_body1,
    grid=(SLAB // TC_BLOCK,),
    in_specs=[
        pl.BlockSpec((PRIOR_DIM, TC_BLOCK), lambda i: (0, i)),
        pl.BlockSpec((TC_BLOCK, EMB_DIM), lambda i: (i, 0)),
    ],
    out_specs=pl.BlockSpec((OUT_DIM, TC_BLOCK), lambda i: (0, i)),
    out_shape=jax.ShapeDtypeStruct((OUT_DIM, N_NODES), jnp.float32),
    compiler_params=pltpu.CompilerParams(
        dimension_semantics=("arbitrary",)),
)

_NB1 = SLAB // TC_BLOCK

_CONCAT2 = pl.pallas_call(
    _concat_body2,
    grid=((SLAB2 + TC_BLOCK - 1) // TC_BLOCK,),
    in_specs=[
        pl.BlockSpec(memory_space=pl.ANY),
        pl.BlockSpec((PRIOR_DIM, TC_BLOCK), lambda i: (0, i + _NB1)),
        pl.BlockSpec((TC_BLOCK, EMB_DIM), lambda i: (i, 0)),
    ],
    out_specs=pl.BlockSpec((OUT_DIM, TC_BLOCK), lambda i: (0, i + _NB1)),
    out_shape=jax.ShapeDtypeStruct((OUT_DIM, N_NODES), jnp.float32),
    input_output_aliases={0: 0},
    compiler_params=pltpu.CompilerParams(
        dimension_semantics=("arbitrary",)),
)


def kernel(feature, invariant_node_features, table):
    feat = feature.astype(jnp.int32)
    inv_t = invariant_node_features.T
    emb1 = _GATHER1(feat[:SLAB], table)
    emb2 = _GATHER2(feat[SLAB:], table)
    out_t = _CONCAT1(inv_t, emb1)
    out_t = _CONCAT2(out_t, inv_t, emb2)
    return out_t.T


# speedup vs baseline: 1.0395x; 1.0395x over previous
import functools

import jax
import jax.numpy as jnp
from jax import lax
from jax.experimental import pallas as pl
from jax.experimental.pallas import tpu as pltpu
from jax.experimental.pallas import tpu_sc as plsc

N_NODES = 100000
EMB_DIM = 128
PRIOR_DIM = 64
OUT_DIM = PRIOR_DIM + EMB_DIM
CHUNK = 384
HALF = 128
NUM_FULL = N_NODES // CHUNK
REM = N_NODES - NUM_FULL * CHUNK
NW = 32
NMAX = (NUM_FULL + NW - 1) // NW
LAST_FULL_W = (NUM_FULL - 1) % NW

TC_BLOCK = 8192


def _build_gather():
    mesh = plsc.VectorSubcoreMesh(core_axis_name="c", subcore_axis_name="s")

    @functools.partial(
        pl.kernel,
        mesh=mesh,
        out_type=jax.ShapeDtypeStruct((N_NODES, EMB_DIM), jnp.float32),
        scratch_types=[
            pltpu.VMEM((CHUNK,), jnp.int32),
            pltpu.VMEM((CHUNK,), jnp.int32),
            pltpu.VMEM((CHUNK, EMB_DIM), jnp.float32),
            pltpu.VMEM((CHUNK, EMB_DIM), jnp.float32),
            pltpu.SemaphoreType.DMA,
            pltpu.SemaphoreType.DMA,
            pltpu.SemaphoreType.DMA,
            pltpu.SemaphoreType.DMA,
            pltpu.SemaphoreType.DMA,
            pltpu.SemaphoreType.DMA,
        ],
    )
    def k(feat_hbm, tab_hbm, emb_hbm,
          idx0, idx1, buf0, buf1, gs0, gs1, is0, is1, ws0, ws1):
        idx = (idx0, idx1)
        buf = (buf0, buf1)
        gsem = (gs0, gs1)
        isem = (is0, is1)
        wsem = (ws0, ws1)
        cid = lax.axis_index("c")
        sid = lax.axis_index("s")
        wid = sid * 2 + cid
        n = jnp.where(wid <= LAST_FULL_W, NMAX, NMAX - 1)

        def issue_gather(s):
            for h in range(CHUNK // HALF):
                pltpu.async_copy(tab_hbm.at[idx[s].at[pl.ds(h * HALF, HALF)]],
                                 buf[s].at[pl.ds(h * HALF, HALF), :], gsem[s])

        def wait_gather(s):
            for h in range(CHUNK // HALF):
                pltpu.make_async_copy(
                    tab_hbm.at[idx[s].at[pl.ds(h * HALF, HALF)]],
                    buf[s].at[pl.ds(h * HALF, HALF), :], gsem[s]).wait()

        def issue_idx(t, s):
            pltpu.async_copy(feat_hbm.at[pl.ds((wid + NW * t) * CHUNK, CHUNK)],
                             idx[s], isem[s])

        def wait_idx(s):
            pltpu.make_async_copy(feat_hbm.at[pl.ds(0, CHUNK)], idx[s],
                                  isem[s]).wait()

        def issue_write(t, s):
            c = wid + NW * t
            pltpu.async_copy(buf[s], emb_hbm.at[pl.ds(c * CHUNK, CHUNK), :],
                             wsem[s])

        def wait_write(s):
            pltpu.make_async_copy(buf[s], emb_hbm.at[pl.ds(0, CHUNK), :],
                                  wsem[s]).wait()

        pltpu.sync_copy(feat_hbm.at[pl.ds(wid * CHUNK, CHUNK)], idx[0])
        issue_gather(0)
        issue_idx(1, 1)

        def half(cur, t_cur):
            nxt = 1 - cur
            t_nxt = t_cur + 1

            @pl.when(t_nxt < n)
            def _():
                wait_idx(nxt)

                @pl.when(t_nxt >= 2)
                def _():
                    wait_write(nxt)

                issue_gather(nxt)

            @pl.when(t_cur < n)
            def _():
                wait_gather(cur)

                @pl.when(t_cur + 2 < n)
                def _():
                    issue_idx(t_cur + 2, cur)

                issue_write(t_cur, cur)

        def body(p, carry):
            half(0, 2 * p)
            half(1, 2 * p + 1)
            return carry

        lax.fori_loop(0, (NMAX + 1) // 2, body, 0)

        wait_write(0)
        wait_write(1)

        @pl.when(wid == NW - 1)
        def _tail():
            base = NUM_FULL * CHUNK
            pltpu.sync_copy(feat_hbm.at[pl.ds(base, HALF)],
                            idx[0].at[pl.ds(0, HALF)])
            pltpu.sync_copy(feat_hbm.at[pl.ds(base + HALF, REM - HALF)],
                            idx[0].at[pl.ds(HALF, REM - HALF)])
            pltpu.async_copy(tab_hbm.at[idx[0].at[pl.ds(0, HALF)]],
                             buf[0].at[pl.ds(0, HALF), :], gsem[0])
            pltpu.async_copy(tab_hbm.at[idx[0].at[pl.ds(HALF, REM - HALF)]],
                             buf[0].at[pl.ds(HALF, REM - HALF), :], gsem[0])
            pltpu.make_async_copy(tab_hbm.at[idx[0].at[pl.ds(0, HALF)]],
                                  buf[0].at[pl.ds(0, HALF), :],
                                  gsem[0]).wait()
            pltpu.make_async_copy(tab_hbm.at[idx[0].at[pl.ds(HALF,
                                                             REM - HALF)]],
                                  buf[0].at[pl.ds(HALF, REM - HALF), :],
                                  gsem[0]).wait()
            pltpu.sync_copy(buf[0].at[pl.ds(0, REM), :],
                            emb_hbm.at[pl.ds(base, REM), :])

    return k


_GATHER = _build_gather()


def _concat_body(inv_ref, emb_ref, out_ref):
    out_ref[:PRIOR_DIM, :] = inv_ref[...]
    out_ref[PRIOR_DIM:, :] = emb_ref[...].T


_CONCAT = pl.pallas_call(
    _concat_body,
    grid=((N_NODES + TC_BLOCK - 1) // TC_BLOCK,),
    in_specs=[
        pl.BlockSpec((PRIOR_DIM, TC_BLOCK), lambda i: (0, i)),
        pl.BlockSpec((TC_BLOCK, EMB_DIM), lambda i: (i, 0)),
    ],
    out_specs=pl.BlockSpec((OUT_DIM, TC_BLOCK), lambda i: (0, i)),
    out_shape=jax.ShapeDtypeStruct((OUT_DIM, N_NODES), jnp.float32),
    compiler_params=pltpu.CompilerParams(
        dimension_semantics=("arbitrary",)),
)


def kernel(feature, invariant_node_features, table):
    feat = feature.astype(jnp.int32)
    emb = _GATHER(feat, table)
    out_t = _CONCAT(invariant_node_features.T, emb)
    return out_t.T


# speedup vs baseline: 1.0442x; 1.0045x over previous
import functools

import jax
import jax.numpy as jnp
from jax import lax
from jax.experimental import pallas as pl
from jax.experimental.pallas import tpu as pltpu
from jax.experimental.pallas import tpu_sc as plsc

N_NODES = 100000
EMB_DIM = 128
PRIOR_DIM = 64
OUT_DIM = PRIOR_DIM + EMB_DIM
CHUNK = 384
HALF = 128
NUM_FULL = N_NODES // CHUNK
REM = N_NODES - NUM_FULL * CHUNK
NW = 32
NMAX = (NUM_FULL + NW - 1) // NW
LAST_FULL_W = (NUM_FULL - 1) % NW

TC_BLOCK = 12288


def _build_gather():
    mesh = plsc.VectorSubcoreMesh(core_axis_name="c", subcore_axis_name="s")

    @functools.partial(
        pl.kernel,
        mesh=mesh,
        out_type=jax.ShapeDtypeStruct((N_NODES, EMB_DIM), jnp.float32),
        scratch_types=[
            pltpu.VMEM((CHUNK,), jnp.int32),
            pltpu.VMEM((CHUNK,), jnp.int32),
            pltpu.VMEM((CHUNK, EMB_DIM), jnp.float32),
            pltpu.VMEM((CHUNK, EMB_DIM), jnp.float32),
            pltpu.SemaphoreType.DMA,
            pltpu.SemaphoreType.DMA,
            pltpu.SemaphoreType.DMA,
            pltpu.SemaphoreType.DMA,
            pltpu.SemaphoreType.DMA,
            pltpu.SemaphoreType.DMA,
        ],
    )
    def k(feat_hbm, tab_hbm, emb_hbm,
          idx0, idx1, buf0, buf1, gs0, gs1, is0, is1, ws0, ws1):
        idx = (idx0, idx1)
        buf = (buf0, buf1)
        gsem = (gs0, gs1)
        isem = (is0, is1)
        wsem = (ws0, ws1)
        cid = lax.axis_index("c")
        sid = lax.axis_index("s")
        wid = sid * 2 + cid
        n = jnp.where(wid <= LAST_FULL_W, NMAX, NMAX - 1)

        def issue_gather(s):
            for h in range(CHUNK // HALF):
                pltpu.async_copy(tab_hbm.at[idx[s].at[pl.ds(h * HALF, HALF)]],
                                 buf[s].at[pl.ds(h * HALF, HALF), :], gsem[s])

        def wait_gather(s):
            for h in range(CHUNK // HALF):
                pltpu.make_async_copy(
                    tab_hbm.at[idx[s].at[pl.ds(h * HALF, HALF)]],
                    buf[s].at[pl.ds(h * HALF, HALF), :], gsem[s]).wait()

        def issue_idx(t, s):
            pltpu.async_copy(feat_hbm.at[pl.ds((wid + NW * t) * CHUNK, CHUNK)],
                             idx[s], isem[s])

        def wait_idx(s):
            pltpu.make_async_copy(feat_hbm.at[pl.ds(0, CHUNK)], idx[s],
                                  isem[s]).wait()

        def issue_write(t, s):
            c = wid + NW * t
            pltpu.async_copy(buf[s], emb_hbm.at[pl.ds(c * CHUNK, CHUNK), :],
                             wsem[s])

        def wait_write(s):
            pltpu.make_async_copy(buf[s], emb_hbm.at[pl.ds(0, CHUNK), :],
                                  wsem[s]).wait()

        pltpu.sync_copy(feat_hbm.at[pl.ds(wid * CHUNK, CHUNK)], idx[0])
        issue_gather(0)
        issue_idx(1, 1)

        def half(cur, t_cur):
            nxt = 1 - cur
            t_nxt = t_cur + 1

            @pl.when(t_nxt < n)
            def _():
                wait_idx(nxt)

                @pl.when(t_nxt >= 2)
                def _():
                    wait_write(nxt)

                issue_gather(nxt)

            @pl.when(t_cur < n)
            def _():
                wait_gather(cur)

                @pl.when(t_cur + 2 < n)
                def _():
                    issue_idx(t_cur + 2, cur)

                issue_write(t_cur, cur)

        def body(p, carry):
            half(0, 2 * p)
            half(1, 2 * p + 1)
            return carry

        lax.fori_loop(0, (NMAX + 1) // 2, body, 0)

        wait_write(0)
        wait_write(1)

        _parts = []
        _off = 0
        while _off < REM:
            _parts.append((_off, min(HALF, REM - _off)))
            _off += HALF

        @pl.when(wid == NW - 1)
        def _tail():
            base = NUM_FULL * CHUNK
            pltpu.sync_copy(feat_hbm.at[pl.ds(base, REM)],
                            idx[0].at[pl.ds(0, REM)])
            for o, sz in _parts:
                pltpu.async_copy(tab_hbm.at[idx[0].at[pl.ds(o, sz)]],
                                 buf[0].at[pl.ds(o, sz), :], gsem[0])
            for o, sz in _parts:
                pltpu.make_async_copy(tab_hbm.at[idx[0].at[pl.ds(o, sz)]],
                                      buf[0].at[pl.ds(o, sz), :],
                                      gsem[0]).wait()
            pltpu.sync_copy(buf[0].at[pl.ds(0, REM), :],
                            emb_hbm.at[pl.ds(base, REM), :])

    return k


_GATHER = _build_gather()


def _concat_body(inv_ref, emb_ref, out_ref):
    out_ref[:PRIOR_DIM, :] = inv_ref[...]
    out_ref[PRIOR_DIM:, :] = emb_ref[...].T


_CONCAT = pl.pallas_call(
    _concat_body,
    grid=((N_NODES + TC_BLOCK - 1) // TC_BLOCK,),
    in_specs=[
        pl.BlockSpec((PRIOR_DIM, TC_BLOCK), lambda i: (0, i)),
        pl.BlockSpec((TC_BLOCK, EMB_DIM), lambda i: (i, 0)),
    ],
    out_specs=pl.BlockSpec((OUT_DIM, TC_BLOCK), lambda i: (0, i)),
    out_shape=jax.ShapeDtypeStruct((OUT_DIM, N_NODES), jnp.float32),
    compiler_params=pltpu.CompilerParams(
        dimension_semantics=("arbitrary",)),
)


def kernel(feature, invariant_node_features, table):
    feat = feature.astype(jnp.int32)
    emb = _GATHER(feat, table)
    out_t = _CONCAT(invariant_node_features.T, emb)
    return out_t.T


# speedup vs baseline: 1.0510x; 1.0066x over previous
import functools

import jax
import jax.numpy as jnp
from jax import lax
from jax.experimental import pallas as pl
from jax.experimental.pallas import tpu as pltpu
from jax.experimental.pallas import tpu_sc as plsc

N_NODES = 100000
EMB_DIM = 128
PRIOR_DIM = 64
OUT_DIM = PRIOR_DIM + EMB_DIM
CHUNK = 384
HALF = 128
NUM_FULL = N_NODES // CHUNK
REM = N_NODES - NUM_FULL * CHUNK
NW = 32
NMAX = (NUM_FULL + NW - 1) // NW
LAST_FULL_W = (NUM_FULL - 1) % NW

TC_BLOCK = 16384


def _build_gather():
    mesh = plsc.VectorSubcoreMesh(core_axis_name="c", subcore_axis_name="s")

    @functools.partial(
        pl.kernel,
        mesh=mesh,
        out_type=jax.ShapeDtypeStruct((N_NODES, EMB_DIM), jnp.float32),
        scratch_types=[
            pltpu.VMEM((CHUNK,), jnp.int32),
            pltpu.VMEM((CHUNK,), jnp.int32),
            pltpu.VMEM((CHUNK, EMB_DIM), jnp.float32),
            pltpu.VMEM((CHUNK, EMB_DIM), jnp.float32),
            pltpu.SemaphoreType.DMA,
            pltpu.SemaphoreType.DMA,
            pltpu.SemaphoreType.DMA,
            pltpu.SemaphoreType.DMA,
            pltpu.SemaphoreType.DMA,
            pltpu.SemaphoreType.DMA,
        ],
    )
    def k(feat_hbm, tab_hbm, emb_hbm,
          idx0, idx1, buf0, buf1, gs0, gs1, is0, is1, ws0, ws1):
        idx = (idx0, idx1)
        buf = (buf0, buf1)
        gsem = (gs0, gs1)
        isem = (is0, is1)
        wsem = (ws0, ws1)
        cid = lax.axis_index("c")
        sid = lax.axis_index("s")
        wid = sid * 2 + cid
        n = jnp.where(wid <= LAST_FULL_W, NMAX, NMAX - 1)

        def issue_gather(s):
            for h in range(CHUNK // HALF):
                pltpu.async_copy(tab_hbm.at[idx[s].at[pl.ds(h * HALF, HALF)]],
                                 buf[s].at[pl.ds(h * HALF, HALF), :], gsem[s])

        def wait_gather(s):
            for h in range(CHUNK // HALF):
                pltpu.make_async_copy(
                    tab_hbm.at[idx[s].at[pl.ds(h * HALF, HALF)]],
                    buf[s].at[pl.ds(h * HALF, HALF), :], gsem[s]).wait()

        def issue_idx(t, s):
            pltpu.async_copy(feat_hbm.at[pl.ds((wid + NW * t) * CHUNK, CHUNK)],
                             idx[s], isem[s])

        def wait_idx(s):
            pltpu.make_async_copy(feat_hbm.at[pl.ds(0, CHUNK)], idx[s],
                                  isem[s]).wait()

        def issue_write(t, s):
            c = wid + NW * t
            pltpu.async_copy(buf[s], emb_hbm.at[pl.ds(c * CHUNK, CHUNK), :],
                             wsem[s])

        def wait_write(s):
            pltpu.make_async_copy(buf[s], emb_hbm.at[pl.ds(0, CHUNK), :],
                                  wsem[s]).wait()

        pltpu.sync_copy(feat_hbm.at[pl.ds(wid * CHUNK, CHUNK)], idx[0])
        issue_gather(0)
        issue_idx(1, 1)

        def half(cur, t_cur):
            nxt = 1 - cur
            t_nxt = t_cur + 1

            @pl.when(t_nxt < n)
            def _():
                wait_idx(nxt)

                @pl.when(t_nxt >= 2)
                def _():
                    wait_write(nxt)

                issue_gather(nxt)

            @pl.when(t_cur < n)
            def _():
                wait_gather(cur)

                @pl.when(t_cur + 2 < n)
                def _():
                    issue_idx(t_cur + 2, cur)

                issue_write(t_cur, cur)

        def body(p, carry):
            half(0, 2 * p)
            half(1, 2 * p + 1)
            return carry

        lax.fori_loop(0, (NMAX + 1) // 2, body, 0)

        wait_write(0)
        wait_write(1)

        _parts = []
        _off = 0
        while _off < REM:
            _parts.append((_off, min(HALF, REM - _off)))
            _off += HALF

        @pl.when(wid == NW - 1)
        def _tail():
            base = NUM_FULL * CHUNK
            pltpu.sync_copy(feat_hbm.at[pl.ds(base, REM)],
                            idx[0].at[pl.ds(0, REM)])
            for o, sz in _parts:
                pltpu.async_copy(tab_hbm.at[idx[0].at[pl.ds(o, sz)]],
                                 buf[0].at[pl.ds(o, sz), :], gsem[0])
            for o, sz in _parts:
                pltpu.make_async_copy(tab_hbm.at[idx[0].at[pl.ds(o, sz)]],
                                      buf[0].at[pl.ds(o, sz), :],
                                      gsem[0]).wait()
            pltpu.sync_copy(buf[0].at[pl.ds(0, REM), :],
                            emb_hbm.at[pl.ds(base, REM), :])

    return k


_GATHER = _build_gather()


def _concat_body(inv_ref, emb_ref, out_ref):
    out_ref[:PRIOR_DIM, :] = inv_ref[...]
    out_ref[PRIOR_DIM:, :] = emb_ref[...].T


_CONCAT = pl.pallas_call(
    _concat_body,
    grid=((N_NODES + TC_BLOCK - 1) // TC_BLOCK,),
    in_specs=[
        pl.BlockSpec((PRIOR_DIM, TC_BLOCK), lambda i: (0, i)),
        pl.BlockSpec((TC_BLOCK, EMB_DIM), lambda i: (i, 0)),
    ],
    out_specs=pl.BlockSpec((OUT_DIM, TC_BLOCK), lambda i: (0, i)),
    out_shape=jax.ShapeDtypeStruct((OUT_DIM, N_NODES), jnp.float32),
    compiler_params=pltpu.CompilerParams(
        dimension_semantics=("arbitrary",)),
)


def kernel(feature, invariant_node_features, table):
    feat = feature.astype(jnp.int32)
    emb = _GATHER(feat, table)
    out_t = _CONCAT(invariant_node_features.T, emb)
    return out_t.T
